# baseline (speedup 1.0000x reference)
import functools

import jax
import jax.numpy as jnp
from jax import lax
from jax.experimental import pallas as pl
from jax.experimental.pallas import tpu as pltpu
from jax.experimental.pallas import tpu_sc as plsc

_NPOINT = 1024
_RADIUS = 0.25
_NSAMPLE = 64
_B, _N, _C = 8, 8192, 128


def _fps_body(x_ref, y_ref, z_ref, nx_ref, ny_ref, nz_ref, dist_ref):
    x = x_ref[...]
    y = y_ref[...]
    z = z_ref[...]
    lanes = jax.lax.broadcasted_iota(jnp.int32, (_B, _N), 1)
    slanes = jax.lax.broadcasted_iota(jnp.int32, (_B, _NPOINT), 1)
    dist_ref[...] = jnp.full((_B, _N), 1e10, jnp.float32)

    def body(i, carry):
        cx, cy, cz = carry
        sel = slanes == i
        nx_ref[...] = jnp.where(sel, cx, nx_ref[...])
        ny_ref[...] = jnp.where(sel, cy, ny_ref[...])
        nz_ref[...] = jnp.where(sel, cz, nz_ref[...])
        dx = x - cx
        dy = y - cy
        dz = z - cz
        d = (dx * dx + dy * dy) + dz * dz
        dn = jnp.minimum(dist_ref[...], d)
        dist_ref[...] = dn
        m = jnp.max(dn, axis=1, keepdims=True)
        fi = jnp.min(jnp.where(dn == m, lanes, _N), axis=1, keepdims=True)
        oh = lanes == fi
        ncx = jnp.sum(jnp.where(oh, x, 0.0), axis=1, keepdims=True)
        ncy = jnp.sum(jnp.where(oh, y, 0.0), axis=1, keepdims=True)
        ncz = jnp.sum(jnp.where(oh, z, 0.0), axis=1, keepdims=True)
        return ncx, ncy, ncz

    c0 = (x[:, 0:1], y[:, 0:1], z[:, 0:1])
    jax.lax.fori_loop(0, _NPOINT, body, c0)


def _fps(xyz):
    nx, ny, nz = pl.pallas_call(
        _fps_body,
        out_shape=[jax.ShapeDtypeStruct((_B, _NPOINT), jnp.float32)] * 3,
        scratch_shapes=[pltpu.VMEM((_B, _N), jnp.float32)],
    )(xyz[:, :, 0], xyz[:, :, 1], xyz[:, :, 2])
    return jnp.stack([nx, ny, nz], axis=-1)


def _g_body(xyz_ref, f_ref, w1x_ref, w1f_ref, g_ref):
    g = jnp.dot(xyz_ref[0], w1x_ref[...], preferred_element_type=jnp.float32)
    g += jnp.dot(f_ref[0], w1f_ref[...], preferred_element_type=jnp.float32)
    g_ref[0] = g


def _g_table(xyz, features, W1):
    w1x = W1[:3]
    w1f = W1[3:]
    blk = 1024
    return pl.pallas_call(
        _g_body,
        grid=(_B, _N // blk),
        in_specs=[
            pl.BlockSpec((1, blk, 3), lambda b, i: (b, i, 0)),
            pl.BlockSpec((1, blk, _C), lambda b, i: (b, i, 0)),
            pl.BlockSpec((3, _C), lambda b, i: (0, 0)),
            pl.BlockSpec((_C, _C), lambda b, i: (0, 0)),
        ],
        out_specs=pl.BlockSpec((1, blk, _C), lambda b, i: (b, i, 0)),
        out_shape=jax.ShapeDtypeStruct((_B, _N, _C), jnp.float32),
    )(xyz, features, w1x, w1f)


_STILE = 128
_SCHUNK = 32


def _mlp_body(g_ref, nxyz_ref, w1x_ref, b1_ref, w2_ref, b2_ref, w3_ref,
              b3_ref, out_ref):
    cadj = jnp.dot(nxyz_ref[0], w1x_ref[...],
                   preferred_element_type=jnp.float32) - b1_ref[...]
    w2 = w2_ref[...]
    b2 = b2_ref[...]
    w3 = w3_ref[...]
    b3 = b3_ref[...]
    rows = _SCHUNK * _NSAMPLE
    pooled = []
    for c in range(_STILE // _SCHUNK):
        g3 = g_ref[0, pl.ds(c * rows, rows), :].reshape(_SCHUNK, _NSAMPLE, _C)
        ca = cadj[c * _SCHUNK:(c + 1) * _SCHUNK, :]
        h1 = jnp.maximum(g3 - ca[:, None, :], 0.0).reshape(rows, _C)
        h2 = jnp.maximum(
            jnp.dot(h1, w2, preferred_element_type=jnp.float32) + b2, 0.0)
        h3 = jnp.maximum(
            jnp.dot(h2, w3, preferred_element_type=jnp.float32) + b3, 0.0)
        pooled.append(jnp.max(h3.reshape(_SCHUNK, _NSAMPLE, 512), axis=1))
    out_ref[0] = jnp.concatenate(pooled, axis=0).T


def _mlp(gathered, new_xyz, W1, b1, W2, b2, W3, b3):
    w1x = W1[:3]
    g = gathered.reshape(_B, _NPOINT * _NSAMPLE, _C)
    return pl.pallas_call(
        _mlp_body,
        grid=(_B, _NPOINT // _STILE),
        in_specs=[
            pl.BlockSpec((1, _STILE * _NSAMPLE, _C), lambda b, i: (b, i, 0)),
            pl.BlockSpec((1, _STILE, 3), lambda b, i: (b, i, 0)),
            pl.BlockSpec((3, _C), lambda b, i: (0, 0)),
            pl.BlockSpec((1, _C), lambda b, i: (0, 0)),
            pl.BlockSpec((_C, 256), lambda b, i: (0, 0)),
            pl.BlockSpec((1, 256), lambda b, i: (0, 0)),
            pl.BlockSpec((256, 512), lambda b, i: (0, 0)),
            pl.BlockSpec((1, 512), lambda b, i: (0, 0)),
        ],
        out_specs=pl.BlockSpec((1, 512, _STILE), lambda b, i: (b, 0, i)),
        out_shape=jax.ShapeDtypeStruct((_B, 512, _NPOINT), jnp.float32),
    )(g, new_xyz, w1x, b1.reshape(1, _C), W2, b2.reshape(1, 256), W3,
      b3.reshape(1, 512))


_NCORES, _NSUB = 2, 16
_NW = _NCORES * _NSUB
_WPB = _NW // _B
_CPW = _NPOINT // _WPB
_NCHUNK = _N // 16


def _bf16_rne(v):
    i = plsc.bitcast(v, jnp.int32)
    r = (i + 0x7FFF + ((i >> 16) & 1)) & ~0xFFFF
    return plsc.bitcast(r, jnp.float32)


_GRP = 4
_NGRP = _CPW // _GRP


def _bqg_body(x_hbm, y_hbm, z_hbm, nx_hbm, ny_hbm, nz_hbm, tab_hbm, out_hbm,
              x_v, y_v, z_v, nx_v, ny_v, nz_v, s2_v, ibuf, idx0, idx1, rows0,
              rows1, gsem0, gsem1, wsem0, wsem1):
    wid = lax.axis_index("s") * _NCORES + lax.axis_index("c")
    b = wid // _WPB
    q = wid % _WPB
    pltpu.sync_copy(x_hbm.at[pl.ds(b * _N, _N)], x_v)
    pltpu.sync_copy(y_hbm.at[pl.ds(b * _N, _N)], y_v)
    pltpu.sync_copy(z_hbm.at[pl.ds(b * _N, _N)], z_v)
    pltpu.sync_copy(nx_hbm.at[pl.ds(b * _NPOINT, _NPOINT)], nx_v)
    pltpu.sync_copy(ny_hbm.at[pl.ds(b * _NPOINT, _NPOINT)], ny_v)
    pltpu.sync_copy(nz_hbm.at[pl.ds(b * _NPOINT, _NPOINT)], nz_v)
    iota16 = lax.broadcasted_iota(jnp.int32, (16,), 0)
    r2 = jnp.float32(_RADIUS * _RADIUS)
    idxg = (idx0, idx1)
    rows = (rows0, rows1)
    gsem = (gsem0, gsem1)
    wsem = (wsem0, wsem1)
    grows = _GRP * _NSAMPLE
    wbase = (b * _NPOINT + q * _CPW) * _NSAMPLE

    def prep(j, _):
        sl = pl.ds(j * 16, 16)
        xs = x_v[sl]
        ys = y_v[sl]
        zs = z_v[sl]
        s2_v[sl] = (xs * xs + ys * ys) + zs * zs
        x_v[sl] = _bf16_rne(xs)
        y_v[sl] = _bf16_rne(ys)
        z_v[sl] = _bf16_rne(zs)
        return 0

    lax.fori_loop(0, _NCHUNK, prep, 0)

    def scan_group(c0, ib):
        cxb, cyb, czb, sc2 = [], [], [], []
        for cc in range(_GRP):
            cvec = jnp.full((16,), c0 + cc, jnp.int32)
            cx = plsc.load_gather(nx_v, [cvec])
            cy = plsc.load_gather(ny_v, [cvec])
            cz = plsc.load_gather(nz_v, [cvec])
            sc2.append((cx * cx + cy * cy) + cz * cz)
            cxb.append(_bf16_rne(cx))
            cyb.append(_bf16_rne(cy))
            czb.append(_bf16_rne(cz))

        def blk_cond(st):
            jb, cs = st[0], st[1]
            return (jb < _NCHUNK // 8) & (cs < _NSAMPLE)

        def blk(st):
            jb = st[0]
            cvs = list(st[2:])
            for u in range(8):
                off = jb * 128 + u * 16
                sl = pl.ds(off, 16)
                xs = x_v[sl]
                ys = y_v[sl]
                zs = z_v[sl]
                pos = off + iota16
                for cc in range(_GRP):
                    dot = (cxb[cc] * xs + cyb[cc] * ys) + czb[cc] * zs
                    sqr = (sc2[cc] + s2_v[sl]) - 2.0 * dot
                    m = sqr <= r2
                    cum = plsc.cumsum(m.astype(jnp.int32))
                    slot = jnp.minimum(cvs[cc] + (cum - 1), 111) + cc * 112
                    plsc.store_scatter(ibuf, [slot], pos, mask=m)
                    pc = plsc.all_reduce_population_count(m)
                    if pc.ndim == 0:
                        pc = jnp.full((16,), pc, jnp.int32)
                    cvs[cc] = cvs[cc] + pc
            mn = jnp.minimum(jnp.minimum(cvs[0], cvs[1]),
                             jnp.minimum(cvs[2], cvs[3]))
            return (jb + 1, jnp.max(mn)) + tuple(cvs)

        zero16 = jnp.zeros((16,), jnp.int32)
        st = lax.while_loop(
            blk_cond, blk,
            (jnp.int32(0), jnp.int32(0)) + (zero16,) * _GRP)
        gbase = b * _N
        for cc in range(_GRP):
            cnt_v = st[2 + cc]
            first = plsc.load_gather(ibuf, [jnp.full((16,), cc * 112,
                                                     jnp.int32)])
            for k in range(_NSAMPLE // 16):
                vals = ibuf[cc * 112 + k * 16:cc * 112 + (k + 1) * 16]
                posk = k * 16 + iota16
                sel = jnp.where(posk < cnt_v, vals, first) + gbase
                ib[pl.ds(cc * _NSAMPLE + k * 16, 16)] = sel

    def superbody(sp, _):
        for s in range(2):
            g = sp * 2 + s
            scan_group(q * _CPW + g * _GRP, idxg[s])

            @pl.when(sp >= 1)
            def _wait_wb():
                pltpu.make_async_copy(
                    rows[s], out_hbm.at[pl.ds(0, grows)], wsem[s]).wait()

            pltpu.async_copy(tab_hbm.at[idxg[s]], rows[s], gsem[s])

            if s == 1:
                pg = g - 1

                def _wb_prev():
                    pltpu.make_async_copy(
                        tab_hbm.at[pl.ds(0, grows)], rows[1 - s],
                        gsem[1 - s]).wait()
                    pltpu.async_copy(
                        rows[1 - s],
                        out_hbm.at[pl.ds(wbase + pg * grows, grows)],
                        wsem[1 - s])

                _wb_prev()
            else:
                pg = g - 1

                @pl.when(sp >= 1)
                def _wb_prev2():
                    pltpu.make_async_copy(
                        tab_hbm.at[pl.ds(0, grows)], rows[1 - s],
                        gsem[1 - s]).wait()
                    pltpu.async_copy(
                        rows[1 - s],
                        out_hbm.at[pl.ds(wbase + pg * grows, grows)],
                        wsem[1 - s])
        return 0

    lax.fori_loop(0, _NGRP // 2, superbody, 0)
    pltpu.make_async_copy(
        tab_hbm.at[pl.ds(0, grows)], rows[1], gsem[1]).wait()
    pltpu.async_copy(
        rows[1], out_hbm.at[pl.ds(wbase + (_NGRP - 1) * grows, grows)],
        wsem[1])
    pltpu.make_async_copy(
        rows[0], out_hbm.at[pl.ds(0, grows)], wsem[0]).wait()
    pltpu.make_async_copy(
        rows[1], out_hbm.at[pl.ds(0, grows)], wsem[1]).wait()


def _bqgather_sc(xyz, new_xyz, table):
    mesh = plsc.VectorSubcoreMesh(core_axis_name="c", subcore_axis_name="s")
    return pl.kernel(
        _bqg_body,
        out_type=jax.ShapeDtypeStruct((_B * _NPOINT * _NSAMPLE, _C),
                                      jnp.float32),
        mesh=mesh,
        compiler_params=pltpu.CompilerParams(needs_layout_passes=False),
        scratch_types=[
            pltpu.VMEM((_N,), jnp.float32),
            pltpu.VMEM((_N,), jnp.float32),
            pltpu.VMEM((_N,), jnp.float32),
            pltpu.VMEM((_NPOINT,), jnp.float32),
            pltpu.VMEM((_NPOINT,), jnp.float32),
            pltpu.VMEM((_NPOINT,), jnp.float32),
            pltpu.VMEM((_N,), jnp.float32),
            pltpu.VMEM((_GRP * 112,), jnp.int32),
            pltpu.VMEM((_GRP * _NSAMPLE,), jnp.int32),
            pltpu.VMEM((_GRP * _NSAMPLE,), jnp.int32),
            pltpu.VMEM((_GRP * _NSAMPLE, _C), jnp.float32),
            pltpu.VMEM((_GRP * _NSAMPLE, _C), jnp.float32),
            pltpu.SemaphoreType.DMA,
            pltpu.SemaphoreType.DMA,
            pltpu.SemaphoreType.DMA,
            pltpu.SemaphoreType.DMA,
        ],
    )(xyz[:, :, 0].reshape(-1), xyz[:, :, 1].reshape(-1),
      xyz[:, :, 2].reshape(-1), new_xyz[:, :, 0].reshape(-1),
      new_xyz[:, :, 1].reshape(-1), new_xyz[:, :, 2].reshape(-1), table)


def kernel(xyz, features, W1, b1, W2, b2, W3, b3):
    new_xyz = _fps(xyz)
    g_tab = _g_table(xyz, features, W1)
    gathered = _bqgather_sc(xyz, new_xyz, g_tab.reshape(_B * _N, _C))
    new_features = _mlp(gathered.reshape(_B, _NPOINT * _NSAMPLE, _C), new_xyz,
                        W1, b1, W2, b2, W3, b3)
    return new_xyz, new_features


# speedup vs baseline: 1.1325x; 1.1325x over previous
import functools

import jax
import jax.numpy as jnp
from jax import lax
from jax.experimental import pallas as pl
from jax.experimental.pallas import tpu as pltpu
from jax.experimental.pallas import tpu_sc as plsc

_NPOINT = 1024
_RADIUS = 0.25
_NSAMPLE = 64
_B, _N, _C = 8, 8192, 128


def _fps_body(x_ref, y_ref, z_ref, nx_ref, ny_ref, nz_ref, dist_ref):
    x = x_ref[...]
    y = y_ref[...]
    z = z_ref[...]
    lanes = jax.lax.broadcasted_iota(jnp.int32, (_B, _N), 1)
    slanes = jax.lax.broadcasted_iota(jnp.int32, (_B, _NPOINT), 1)
    dist_ref[...] = jnp.full((_B, _N), 1e10, jnp.float32)

    def body(i, carry):
        cx, cy, cz = carry
        sel = slanes == i
        nx_ref[...] = jnp.where(sel, cx, nx_ref[...])
        ny_ref[...] = jnp.where(sel, cy, ny_ref[...])
        nz_ref[...] = jnp.where(sel, cz, nz_ref[...])
        dx = x - cx
        dy = y - cy
        dz = z - cz
        d = (dx * dx + dy * dy) + dz * dz
        dn = jnp.minimum(dist_ref[...], d)
        dist_ref[...] = dn
        m = jnp.max(dn, axis=1, keepdims=True)
        fi = jnp.min(jnp.where(dn == m, lanes, _N), axis=1, keepdims=True)
        oh = lanes == fi
        ncx = jnp.sum(jnp.where(oh, x, 0.0), axis=1, keepdims=True)
        ncy = jnp.sum(jnp.where(oh, y, 0.0), axis=1, keepdims=True)
        ncz = jnp.sum(jnp.where(oh, z, 0.0), axis=1, keepdims=True)
        return ncx, ncy, ncz

    c0 = (x[:, 0:1], y[:, 0:1], z[:, 0:1])
    jax.lax.fori_loop(0, _NPOINT, body, c0)


def _fps(xyz):
    nx, ny, nz = pl.pallas_call(
        _fps_body,
        out_shape=[jax.ShapeDtypeStruct((_B, _NPOINT), jnp.float32)] * 3,
        scratch_shapes=[pltpu.VMEM((_B, _N), jnp.float32)],
    )(xyz[:, :, 0], xyz[:, :, 1], xyz[:, :, 2])
    return jnp.stack([nx, ny, nz], axis=-1)


def _g_body(xyz_ref, f_ref, w1x_ref, w1f_ref, g_ref):
    g = jnp.dot(xyz_ref[0], w1x_ref[...], preferred_element_type=jnp.float32)
    g += jnp.dot(f_ref[0], w1f_ref[...], preferred_element_type=jnp.float32)
    g_ref[0] = g


def _g_table(xyz, features, W1):
    w1x = W1[:3]
    w1f = W1[3:]
    blk = 1024
    return pl.pallas_call(
        _g_body,
        grid=(_B, _N // blk),
        in_specs=[
            pl.BlockSpec((1, blk, 3), lambda b, i: (b, i, 0)),
            pl.BlockSpec((1, blk, _C), lambda b, i: (b, i, 0)),
            pl.BlockSpec((3, _C), lambda b, i: (0, 0)),
            pl.BlockSpec((_C, _C), lambda b, i: (0, 0)),
        ],
        out_specs=pl.BlockSpec((1, blk, _C), lambda b, i: (b, i, 0)),
        out_shape=jax.ShapeDtypeStruct((_B, _N, _C), jnp.float32),
    )(xyz, features, w1x, w1f)


_STILE = 128
_SCHUNK = 32


def _mlp_body(g_ref, nxyz_ref, w1x_ref, b1_ref, w2_ref, b2_ref, w3_ref,
              b3_ref, out_ref):
    cadj = jnp.dot(nxyz_ref[0], w1x_ref[...],
                   preferred_element_type=jnp.float32) - b1_ref[...]
    w2 = w2_ref[...]
    b2 = b2_ref[...]
    w3 = w3_ref[...]
    b3 = b3_ref[...]
    rows = _SCHUNK * _NSAMPLE
    pooled = []
    for c in range(_STILE // _SCHUNK):
        g3 = g_ref[0, pl.ds(c * rows, rows), :].reshape(_SCHUNK, _NSAMPLE, _C)
        ca = cadj[c * _SCHUNK:(c + 1) * _SCHUNK, :]
        h1 = jnp.maximum(g3 - ca[:, None, :], 0.0).reshape(rows, _C)
        h2 = jnp.maximum(
            jnp.dot(h1, w2, preferred_element_type=jnp.float32) + b2, 0.0)
        h3 = jnp.maximum(
            jnp.dot(h2, w3, preferred_element_type=jnp.float32) + b3, 0.0)
        pooled.append(jnp.max(h3.reshape(_SCHUNK, _NSAMPLE, 512), axis=1))
    out_ref[0] = jnp.concatenate(pooled, axis=0).T


def _mlp(gathered, new_xyz, W1, b1, W2, b2, W3, b3):
    w1x = W1[:3]
    g = gathered.reshape(_B, _NPOINT * _NSAMPLE, _C)
    return pl.pallas_call(
        _mlp_body,
        grid=(_B, _NPOINT // _STILE),
        in_specs=[
            pl.BlockSpec((1, _STILE * _NSAMPLE, _C), lambda b, i: (b, i, 0)),
            pl.BlockSpec((1, _STILE, 3), lambda b, i: (b, i, 0)),
            pl.BlockSpec((3, _C), lambda b, i: (0, 0)),
            pl.BlockSpec((1, _C), lambda b, i: (0, 0)),
            pl.BlockSpec((_C, 256), lambda b, i: (0, 0)),
            pl.BlockSpec((1, 256), lambda b, i: (0, 0)),
            pl.BlockSpec((256, 512), lambda b, i: (0, 0)),
            pl.BlockSpec((1, 512), lambda b, i: (0, 0)),
        ],
        out_specs=pl.BlockSpec((1, 512, _STILE), lambda b, i: (b, 0, i)),
        out_shape=jax.ShapeDtypeStruct((_B, 512, _NPOINT), jnp.float32),
    )(g, new_xyz, w1x, b1.reshape(1, _C), W2, b2.reshape(1, 256), W3,
      b3.reshape(1, 512))


_NCORES, _NSUB = 2, 16
_NW = _NCORES * _NSUB
_WPB = _NW // _B
_CPW = _NPOINT // _WPB
_NCHUNK = _N // 16


def _bf16_rne(v):
    i = plsc.bitcast(v, jnp.int32)
    r = (i + 0x7FFF + ((i >> 16) & 1)) & ~0xFFFF
    return plsc.bitcast(r, jnp.float32)


_GRP = 4
_NGRP = _CPW // _GRP


def _bqg_body(x_hbm, y_hbm, z_hbm, nx_hbm, ny_hbm, nz_hbm, tab_hbm, out_hbm,
              x_v, y_v, z_v, nx_v, ny_v, nz_v, s2_v, ibuf, idx0, idx1, rows0,
              rows1, gsem0, gsem1, wsem0, wsem1):
    wid = lax.axis_index("s") * _NCORES + lax.axis_index("c")
    b = wid // _WPB
    q = wid % _WPB
    pltpu.sync_copy(x_hbm.at[pl.ds(b * _N, _N)], x_v)
    pltpu.sync_copy(y_hbm.at[pl.ds(b * _N, _N)], y_v)
    pltpu.sync_copy(z_hbm.at[pl.ds(b * _N, _N)], z_v)
    pltpu.sync_copy(nx_hbm.at[pl.ds(b * _NPOINT, _NPOINT)], nx_v)
    pltpu.sync_copy(ny_hbm.at[pl.ds(b * _NPOINT, _NPOINT)], ny_v)
    pltpu.sync_copy(nz_hbm.at[pl.ds(b * _NPOINT, _NPOINT)], nz_v)
    iota16 = lax.broadcasted_iota(jnp.int32, (16,), 0)
    r2 = jnp.float32(_RADIUS * _RADIUS)
    idxg = (idx0, idx1)
    rows = (rows0, rows1)
    gsem = (gsem0, gsem1)
    wsem = (wsem0, wsem1)
    grows = _GRP * _NSAMPLE
    wbase = (b * _NPOINT + q * _CPW) * _NSAMPLE

    def prep(j, _):
        sl = pl.ds(j * 16, 16)
        xs = x_v[sl]
        ys = y_v[sl]
        zs = z_v[sl]
        s2_v[sl] = (xs * xs + ys * ys) + zs * zs
        x_v[sl] = _bf16_rne(xs)
        y_v[sl] = _bf16_rne(ys)
        z_v[sl] = _bf16_rne(zs)
        return 0

    lax.fori_loop(0, _NCHUNK, prep, 0)

    def scan_centroid(c, ib, cc):
        cvec = jnp.full((16,), c, jnp.int32)
        cx = plsc.load_gather(nx_v, [cvec])
        cy = plsc.load_gather(ny_v, [cvec])
        cz = plsc.load_gather(nz_v, [cvec])
        sc2 = (cx * cx + cy * cy) + cz * cz
        cxb = _bf16_rne(cx)
        cyb = _bf16_rne(cy)
        czb = _bf16_rne(cz)

        def blk_cond(st):
            jb, cs, _cv = st
            return (jb < _NCHUNK // 8) & (cs < _NSAMPLE)

        def blk(st):
            jb, cs, cv = st
            for u in range(8):
                off = jb * 128 + u * 16
                sl = pl.ds(off, 16)
                dot = (cxb * x_v[sl] + cyb * y_v[sl]) + czb * z_v[sl]
                sqr = (sc2 + s2_v[sl]) - 2.0 * dot
                m = sqr <= r2
                cum = plsc.cumsum(m.astype(jnp.int32))
                pos = off + iota16
                plsc.store_scatter(ibuf, [cv + (cum - 1)], pos, mask=m)
                pc = plsc.all_reduce_population_count(m)
                if pc.ndim == 0:
                    pc = jnp.full((16,), pc, jnp.int32)
                cv = cv + pc
            return jb + 1, jnp.max(cv), cv

        zero16 = jnp.zeros((16,), jnp.int32)
        _, _, cnt_v = lax.while_loop(
            blk_cond, blk, (jnp.int32(0), jnp.int32(0), zero16))
        first = plsc.load_gather(ibuf, [zero16])
        gbase = b * _N
        for k in range(_NSAMPLE // 16):
            vals = ibuf[k * 16:(k + 1) * 16]
            posk = k * 16 + iota16
            sel = jnp.where(posk < cnt_v, vals, first) + gbase
            ib[pl.ds(cc * _NSAMPLE + k * 16, 16)] = sel

    def superbody(sp, _):
        for s in range(2):
            g = sp * 2 + s
            for cc in range(_GRP):
                scan_centroid(q * _CPW + g * _GRP + cc, idxg[s], cc)

            @pl.when(sp >= 1)
            def _wait_wb():
                pltpu.make_async_copy(
                    rows[s], out_hbm.at[pl.ds(0, grows)], wsem[s]).wait()

            pltpu.async_copy(tab_hbm.at[idxg[s]], rows[s], gsem[s])

            if s == 1:
                pg = g - 1

                def _wb_prev():
                    pltpu.make_async_copy(
                        tab_hbm.at[pl.ds(0, grows)], rows[1 - s],
                        gsem[1 - s]).wait()
                    pltpu.async_copy(
                        rows[1 - s],
                        out_hbm.at[pl.ds(wbase + pg * grows, grows)],
                        wsem[1 - s])

                _wb_prev()
            else:
                pg = g - 1

                @pl.when(sp >= 1)
                def _wb_prev2():
                    pltpu.make_async_copy(
                        tab_hbm.at[pl.ds(0, grows)], rows[1 - s],
                        gsem[1 - s]).wait()
                    pltpu.async_copy(
                        rows[1 - s],
                        out_hbm.at[pl.ds(wbase + pg * grows, grows)],
                        wsem[1 - s])
        return 0

    lax.fori_loop(0, _NGRP // 2, superbody, 0)
    pltpu.make_async_copy(
        tab_hbm.at[pl.ds(0, grows)], rows[1], gsem[1]).wait()
    pltpu.async_copy(
        rows[1], out_hbm.at[pl.ds(wbase + (_NGRP - 1) * grows, grows)],
        wsem[1])
    pltpu.make_async_copy(
        rows[0], out_hbm.at[pl.ds(0, grows)], wsem[0]).wait()
    pltpu.make_async_copy(
        rows[1], out_hbm.at[pl.ds(0, grows)], wsem[1]).wait()


def _bqgather_sc(xyz, new_xyz, table):
    mesh = plsc.VectorSubcoreMesh(core_axis_name="c", subcore_axis_name="s")
    return pl.kernel(
        _bqg_body,
        out_type=jax.ShapeDtypeStruct((_B * _NPOINT * _NSAMPLE, _C),
                                      jnp.float32),
        mesh=mesh,
        compiler_params=pltpu.CompilerParams(needs_layout_passes=False),
        scratch_types=[
            pltpu.VMEM((_N,), jnp.float32),
            pltpu.VMEM((_N,), jnp.float32),
            pltpu.VMEM((_N,), jnp.float32),
            pltpu.VMEM((_NPOINT,), jnp.float32),
            pltpu.VMEM((_NPOINT,), jnp.float32),
            pltpu.VMEM((_NPOINT,), jnp.float32),
            pltpu.VMEM((_N,), jnp.float32),
            pltpu.VMEM((_GRP * 112,), jnp.int32),
            pltpu.VMEM((_GRP * _NSAMPLE,), jnp.int32),
            pltpu.VMEM((_GRP * _NSAMPLE,), jnp.int32),
            pltpu.VMEM((_GRP * _NSAMPLE, _C), jnp.float32),
            pltpu.VMEM((_GRP * _NSAMPLE, _C), jnp.float32),
            pltpu.SemaphoreType.DMA,
            pltpu.SemaphoreType.DMA,
            pltpu.SemaphoreType.DMA,
            pltpu.SemaphoreType.DMA,
        ],
    )(xyz[:, :, 0].reshape(-1), xyz[:, :, 1].reshape(-1),
      xyz[:, :, 2].reshape(-1), new_xyz[:, :, 0].reshape(-1),
      new_xyz[:, :, 1].reshape(-1), new_xyz[:, :, 2].reshape(-1), table)


def kernel(xyz, features, W1, b1, W2, b2, W3, b3):
    new_xyz = _fps(xyz)
    g_tab = _g_table(xyz, features, W1)
    gathered = _bqgather_sc(xyz, new_xyz, g_tab.reshape(_B * _N, _C))
    new_features = _mlp(gathered.reshape(_B, _NPOINT * _NSAMPLE, _C), new_xyz,
                        W1, b1, W2, b2, W3, b3)
    return new_xyz, new_features


# speedup vs baseline: 1.5476x; 1.3665x over previous
import functools

import jax
import jax.numpy as jnp
from jax import lax
from jax.experimental import pallas as pl
from jax.experimental.pallas import tpu as pltpu
from jax.experimental.pallas import tpu_sc as plsc

_NPOINT = 1024
_RADIUS = 0.25
_NSAMPLE = 64
_B, _N, _C = 8, 8192, 128


def _fps_body(x_ref, y_ref, z_ref, nx_ref, ny_ref, nz_ref, dist_ref):
    x = x_ref[...]
    y = y_ref[...]
    z = z_ref[...]
    lanes = jax.lax.broadcasted_iota(jnp.int32, (_B, _N), 1)
    slanes = jax.lax.broadcasted_iota(jnp.int32, (_B, _NPOINT), 1)
    dist_ref[...] = jnp.full((_B, _N), 1e10, jnp.float32)

    def body(i, carry):
        cx, cy, cz = carry
        sel = slanes == i
        nx_ref[...] = jnp.where(sel, cx, nx_ref[...])
        ny_ref[...] = jnp.where(sel, cy, ny_ref[...])
        nz_ref[...] = jnp.where(sel, cz, nz_ref[...])
        dx = x - cx
        dy = y - cy
        dz = z - cz
        d = (dx * dx + dy * dy) + dz * dz
        dn = jnp.minimum(dist_ref[...], d)
        dist_ref[...] = dn
        m = jnp.max(dn, axis=1, keepdims=True)
        fi = jnp.min(jnp.where(dn == m, lanes, _N), axis=1, keepdims=True)
        oh = lanes == fi
        ncx = jnp.sum(jnp.where(oh, x, 0.0), axis=1, keepdims=True)
        ncy = jnp.sum(jnp.where(oh, y, 0.0), axis=1, keepdims=True)
        ncz = jnp.sum(jnp.where(oh, z, 0.0), axis=1, keepdims=True)
        return ncx, ncy, ncz

    c0 = (x[:, 0:1], y[:, 0:1], z[:, 0:1])
    jax.lax.fori_loop(0, _NPOINT, body, c0)


def _fps(xyz):
    nx, ny, nz = pl.pallas_call(
        _fps_body,
        out_shape=[jax.ShapeDtypeStruct((_B, _NPOINT), jnp.float32)] * 3,
        scratch_shapes=[pltpu.VMEM((_B, _N), jnp.float32)],
    )(xyz[:, :, 0], xyz[:, :, 1], xyz[:, :, 2])
    return jnp.stack([nx, ny, nz], axis=-1)


def _g_body(xyz_ref, f_ref, w1x_ref, w1f_ref, g_ref):
    g = jnp.dot(xyz_ref[0], w1x_ref[...], preferred_element_type=jnp.float32)
    g += jnp.dot(f_ref[0], w1f_ref[...], preferred_element_type=jnp.float32)
    g_ref[0] = g


def _g_table(xyz, features, W1):
    w1x = W1[:3]
    w1f = W1[3:]
    blk = 1024
    return pl.pallas_call(
        _g_body,
        grid=(_B, _N // blk),
        in_specs=[
            pl.BlockSpec((1, blk, 3), lambda b, i: (b, i, 0)),
            pl.BlockSpec((1, blk, _C), lambda b, i: (b, i, 0)),
            pl.BlockSpec((3, _C), lambda b, i: (0, 0)),
            pl.BlockSpec((_C, _C), lambda b, i: (0, 0)),
        ],
        out_specs=pl.BlockSpec((1, blk, _C), lambda b, i: (b, i, 0)),
        out_shape=jax.ShapeDtypeStruct((_B, _N, _C), jnp.float32),
    )(xyz, features, w1x, w1f)


_STILE = 128
_SCHUNK = 32


def _mlp_body(g_ref, nxyz_ref, w1x_ref, b1_ref, w2_ref, b2_ref, w3_ref,
              b3_ref, out_ref):
    cadj = jnp.dot(nxyz_ref[0], w1x_ref[...],
                   preferred_element_type=jnp.float32) - b1_ref[...]
    w2 = w2_ref[...]
    b2 = b2_ref[...]
    w3 = w3_ref[...]
    b3 = b3_ref[...]
    rows = _SCHUNK * _NSAMPLE
    pooled = []
    for c in range(_STILE // _SCHUNK):
        g3 = g_ref[0, pl.ds(c * rows, rows), :].reshape(_SCHUNK, _NSAMPLE, _C)
        ca = cadj[c * _SCHUNK:(c + 1) * _SCHUNK, :]
        h1 = jnp.maximum(g3 - ca[:, None, :], 0.0).reshape(rows, _C)
        h2 = jnp.maximum(
            jnp.dot(h1, w2, preferred_element_type=jnp.float32) + b2, 0.0)
        h3 = jnp.maximum(
            jnp.dot(h2, w3, preferred_element_type=jnp.float32) + b3, 0.0)
        pooled.append(jnp.max(h3.reshape(_SCHUNK, _NSAMPLE, 512), axis=1))
    out_ref[0] = jnp.concatenate(pooled, axis=0).T


def _mlp(gathered, new_xyz, W1, b1, W2, b2, W3, b3):
    w1x = W1[:3]
    g = gathered.reshape(_B, _NPOINT * _NSAMPLE, _C)
    return pl.pallas_call(
        _mlp_body,
        grid=(_B, _NPOINT // _STILE),
        in_specs=[
            pl.BlockSpec((1, _STILE * _NSAMPLE, _C), lambda b, i: (b, i, 0)),
            pl.BlockSpec((1, _STILE, 3), lambda b, i: (b, i, 0)),
            pl.BlockSpec((3, _C), lambda b, i: (0, 0)),
            pl.BlockSpec((1, _C), lambda b, i: (0, 0)),
            pl.BlockSpec((_C, 256), lambda b, i: (0, 0)),
            pl.BlockSpec((1, 256), lambda b, i: (0, 0)),
            pl.BlockSpec((256, 512), lambda b, i: (0, 0)),
            pl.BlockSpec((1, 512), lambda b, i: (0, 0)),
        ],
        out_specs=pl.BlockSpec((1, 512, _STILE), lambda b, i: (b, 0, i)),
        out_shape=jax.ShapeDtypeStruct((_B, 512, _NPOINT), jnp.float32),
    )(g, new_xyz, w1x, b1.reshape(1, _C), W2, b2.reshape(1, 256), W3,
      b3.reshape(1, 512))


_NCORES, _NSUB = 2, 16
_NW = _NCORES * _NSUB
_WPB = _NW // _B
_CPW = _NPOINT // _WPB
_NCHUNK = _N // 16


def _bf16_rne(v):
    i = plsc.bitcast(v, jnp.int32)
    r = (i + 0x7FFF + ((i >> 16) & 1)) & ~0xFFFF
    return plsc.bitcast(r, jnp.float32)


_GRP = 4
_NGRP = _CPW // _GRP


def _bqg_body(x_hbm, y_hbm, z_hbm, nx_hbm, ny_hbm, nz_hbm, tab_hbm, out_hbm,
              x_v, y_v, z_v, nx_v, ny_v, nz_v, s2_v, ibuf, idx0, idx1, rows0,
              rows1, gsem0, gsem1, wsem0, wsem1):
    wid = lax.axis_index("s") * _NCORES + lax.axis_index("c")
    b = wid // _WPB
    q = wid % _WPB
    pltpu.sync_copy(x_hbm.at[pl.ds(b * _N, _N)], x_v)
    pltpu.sync_copy(y_hbm.at[pl.ds(b * _N, _N)], y_v)
    pltpu.sync_copy(z_hbm.at[pl.ds(b * _N, _N)], z_v)
    pltpu.sync_copy(nx_hbm.at[pl.ds(b * _NPOINT, _NPOINT)], nx_v)
    pltpu.sync_copy(ny_hbm.at[pl.ds(b * _NPOINT, _NPOINT)], ny_v)
    pltpu.sync_copy(nz_hbm.at[pl.ds(b * _NPOINT, _NPOINT)], nz_v)
    iota16 = lax.broadcasted_iota(jnp.int32, (16,), 0)
    r2 = jnp.float32(_RADIUS * _RADIUS)
    idxg = (idx0, idx1)
    rows = (rows0, rows1)
    gsem = (gsem0, gsem1)
    wsem = (wsem0, wsem1)
    grows = _GRP * _NSAMPLE
    wbase = (b * _NPOINT + q * _CPW) * _NSAMPLE

    def prep(j, _):
        sl = pl.ds(j * 16, 16)
        xs = x_v[sl]
        ys = y_v[sl]
        zs = z_v[sl]
        s2_v[sl] = (xs * xs + ys * ys) + zs * zs
        x_v[sl] = _bf16_rne(xs)
        y_v[sl] = _bf16_rne(ys)
        z_v[sl] = _bf16_rne(zs)
        return 0

    lax.fori_loop(0, _NCHUNK, prep, 0)

    def scan_centroid(c, ib, cc):
        cvec = jnp.full((16,), c, jnp.int32)
        cx = plsc.load_gather(nx_v, [cvec])
        cy = plsc.load_gather(ny_v, [cvec])
        cz = plsc.load_gather(nz_v, [cvec])
        sc2 = (cx * cx + cy * cy) + cz * cz
        cxb = _bf16_rne(cx)
        cyb = _bf16_rne(cy)
        czb = _bf16_rne(cz)

        def blk_cond(st):
            jb, cs, _cv = st
            return (jb < _NCHUNK // 8) & (cs < _NSAMPLE)

        def blk(st):
            jb, cs, cv = st
            slots, poss, ms = [], [], []
            for u in range(8):
                off = jb * 128 + u * 16
                sl = pl.ds(off, 16)
                dot = (cxb * x_v[sl] + cyb * y_v[sl]) + czb * z_v[sl]
                sqr = (sc2 + s2_v[sl]) - 2.0 * dot
                m = sqr <= r2
                cum = plsc.cumsum(m.astype(jnp.int32))
                slots.append(cv + (cum - 1))
                poss.append(off + iota16)
                ms.append(m)
                pc = plsc.all_reduce_population_count(m)
                if pc.ndim == 0:
                    pc = jnp.full((16,), pc, jnp.int32)
                cv = cv + pc
            for u in range(8):
                plsc.store_scatter(ibuf, [slots[u]], poss[u], mask=ms[u])
            return jb + 1, jnp.max(cv), cv

        zero16 = jnp.zeros((16,), jnp.int32)
        _, _, cnt_v = lax.while_loop(
            blk_cond, blk, (jnp.int32(0), jnp.int32(0), zero16))
        first = plsc.load_gather(ibuf, [zero16])
        gbase = b * _N
        for k in range(_NSAMPLE // 16):
            vals = ibuf[k * 16:(k + 1) * 16]
            posk = k * 16 + iota16
            sel = jnp.where(posk < cnt_v, vals, first) + gbase
            ib[pl.ds(cc * _NSAMPLE + k * 16, 16)] = sel

    def superbody(sp, _):
        for s in range(2):
            g = sp * 2 + s
            for cc in range(_GRP):
                scan_centroid(q * _CPW + g * _GRP + cc, idxg[s], cc)

            @pl.when(sp >= 1)
            def _wait_wb():
                pltpu.make_async_copy(
                    rows[s], out_hbm.at[pl.ds(0, grows)], wsem[s]).wait()

            pltpu.async_copy(tab_hbm.at[idxg[s]], rows[s], gsem[s])

            if s == 1:
                pg = g - 1

                def _wb_prev():
                    pltpu.make_async_copy(
                        tab_hbm.at[pl.ds(0, grows)], rows[1 - s],
                        gsem[1 - s]).wait()
                    pltpu.async_copy(
                        rows[1 - s],
                        out_hbm.at[pl.ds(wbase + pg * grows, grows)],
                        wsem[1 - s])

                _wb_prev()
            else:
                pg = g - 1

                @pl.when(sp >= 1)
                def _wb_prev2():
                    pltpu.make_async_copy(
                        tab_hbm.at[pl.ds(0, grows)], rows[1 - s],
                        gsem[1 - s]).wait()
                    pltpu.async_copy(
                        rows[1 - s],
                        out_hbm.at[pl.ds(wbase + pg * grows, grows)],
                        wsem[1 - s])
        return 0

    lax.fori_loop(0, _NGRP // 2, superbody, 0)
    pltpu.make_async_copy(
        tab_hbm.at[pl.ds(0, grows)], rows[1], gsem[1]).wait()
    pltpu.async_copy(
        rows[1], out_hbm.at[pl.ds(wbase + (_NGRP - 1) * grows, grows)],
        wsem[1])
    pltpu.make_async_copy(
        rows[0], out_hbm.at[pl.ds(0, grows)], wsem[0]).wait()
    pltpu.make_async_copy(
        rows[1], out_hbm.at[pl.ds(0, grows)], wsem[1]).wait()


def _bqgather_sc(xyz, new_xyz, table):
    mesh = plsc.VectorSubcoreMesh(core_axis_name="c", subcore_axis_name="s")
    return pl.kernel(
        _bqg_body,
        out_type=jax.ShapeDtypeStruct((_B * _NPOINT * _NSAMPLE, _C),
                                      jnp.float32),
        mesh=mesh,
        compiler_params=pltpu.CompilerParams(needs_layout_passes=False),
        scratch_types=[
            pltpu.VMEM((_N,), jnp.float32),
            pltpu.VMEM((_N,), jnp.float32),
            pltpu.VMEM((_N,), jnp.float32),
            pltpu.VMEM((_NPOINT,), jnp.float32),
            pltpu.VMEM((_NPOINT,), jnp.float32),
            pltpu.VMEM((_NPOINT,), jnp.float32),
            pltpu.VMEM((_N,), jnp.float32),
            pltpu.VMEM((_GRP * 112,), jnp.int32),
            pltpu.VMEM((_GRP * _NSAMPLE,), jnp.int32),
            pltpu.VMEM((_GRP * _NSAMPLE,), jnp.int32),
            pltpu.VMEM((_GRP * _NSAMPLE, _C), jnp.float32),
            pltpu.VMEM((_GRP * _NSAMPLE, _C), jnp.float32),
            pltpu.SemaphoreType.DMA,
            pltpu.SemaphoreType.DMA,
            pltpu.SemaphoreType.DMA,
            pltpu.SemaphoreType.DMA,
        ],
    )(xyz[:, :, 0].reshape(-1), xyz[:, :, 1].reshape(-1),
      xyz[:, :, 2].reshape(-1), new_xyz[:, :, 0].reshape(-1),
      new_xyz[:, :, 1].reshape(-1), new_xyz[:, :, 2].reshape(-1), table)


def kernel(xyz, features, W1, b1, W2, b2, W3, b3):
    new_xyz = _fps(xyz)
    g_tab = _g_table(xyz, features, W1)
    gathered = _bqgather_sc(xyz, new_xyz, g_tab.reshape(_B * _N, _C))
    new_features = _mlp(gathered.reshape(_B, _NPOINT * _NSAMPLE, _C), new_xyz,
                        W1, b1, W2, b2, W3, b3)
    return new_xyz, new_features
